# baseline (speedup 1.0000x reference)
import jax
import jax.numpy as jnp
from jax import lax
from jax.experimental import pallas as pl
from jax.experimental.pallas import tpu as pltpu

NUM_CODES = 1024
DIM = 64
PIX = 1024


def _vq_body(x_ref, cb_ref, zis_ref, zqs_ref, cbn2_ref, c2_ref, cbt_ref):
    @pl.when(pl.program_id(0) == 0)
    def _init():
        cb0 = cb_ref[...]
        cbn2_ref[...] = cb0 * -2.0
        c2_ref[...] = jnp.sum(cb0 * cb0, axis=1, keepdims=True)
        cbt_ref[...] = cb0.T

    for s in range(x_ref.shape[0]):
        x = x_ref[s]

        mmn2 = lax.dot_general(cbn2_ref[...], x, (((1,), (0,)), ((), ())),
                               precision=lax.Precision.DEFAULT)
        z2 = jnp.sum(x * x, axis=0)
        dist = (z2[None, :] + c2_ref[...]) + mmn2

        idx = jnp.argmin(dist, axis=0)
        ii = lax.broadcasted_iota(jnp.int32, (NUM_CODES, PIX), 0)
        zis_ref[s] = idx.reshape(8, 128)

        onehot = (ii == idx[None, :]).astype(jnp.float32)
        zq = lax.dot_general(cbt_ref[...], onehot, (((1,), (0,)), ((), ())),
                             precision=lax.Precision.DEFAULT)
        zqs_ref[s] = zq


BB = 8


def kernel(inputs, codebook):
    B = inputs.shape[0]
    x = inputs.reshape(B, DIM, PIX)
    zis, zqs = pl.pallas_call(
        _vq_body,
        grid=(B // BB,),
        in_specs=[
            pl.BlockSpec((BB, DIM, PIX), lambda b: (b, 0, 0)),
            pl.BlockSpec((NUM_CODES, DIM), lambda b: (0, 0)),
        ],
        out_specs=[
            pl.BlockSpec((BB, 8, 128), lambda b: (b, 0, 0)),
            pl.BlockSpec((BB, DIM, PIX), lambda b: (b, 0, 0)),
        ],
        out_shape=[
            jax.ShapeDtypeStruct((B, 8, 128), jnp.int32),
            jax.ShapeDtypeStruct((B, DIM, PIX), jnp.float32),
        ],
        scratch_shapes=[
            pltpu.VMEM((NUM_CODES, DIM), jnp.float32),
            pltpu.VMEM((NUM_CODES, 1), jnp.float32),
            pltpu.VMEM((DIM, NUM_CODES), jnp.float32),
        ],
    )(x, codebook)
    return (zis.reshape(B, 32, 32), zqs.reshape(B, DIM, 32, 32))


# speedup vs baseline: 1.0047x; 1.0047x over previous
import jax
import jax.numpy as jnp
from jax import lax
from jax.experimental import pallas as pl
from jax.experimental.pallas import tpu as pltpu

NUM_CODES = 1024
DIM = 64
PIX = 1024


def _vq_body(x_ref, cb_ref, zis_ref, zqs_ref, cbn2_ref, c2_ref, cbt_ref):
    @pl.when(pl.program_id(0) == 0)
    def _init():
        cb0 = cb_ref[...]
        cbn2_ref[...] = cb0 * -2.0
        c2_ref[...] = jnp.sum(cb0 * cb0, axis=1, keepdims=True)
        cbt_ref[...] = cb0.T

    for s in range(x_ref.shape[0]):
        x = x_ref[s]

        mmn2 = lax.dot_general(cbn2_ref[...], x, (((1,), (0,)), ((), ())),
                               precision=lax.Precision.DEFAULT)
        z2 = jnp.sum(x * x, axis=0)
        dist = (z2[None, :] + c2_ref[...]) + mmn2

        idx = jnp.argmin(dist, axis=0)
        ii = lax.broadcasted_iota(jnp.int32, (NUM_CODES, PIX), 0)
        for j in range(32):
            zis_ref[s, j, :] = lax.slice(idx, (32 * j,), (32 * j + 32,))

        onehot = (ii == idx[None, :]).astype(jnp.float32)
        zq = lax.dot_general(cbt_ref[...], onehot, (((1,), (0,)), ((), ())),
                             precision=lax.Precision.DEFAULT)
        zqs_ref[s] = zq


BB = 4


def kernel(inputs, codebook):
    B = inputs.shape[0]
    x = inputs.reshape(B, DIM, PIX)
    zis, zqs = pl.pallas_call(
        _vq_body,
        grid=(B // BB,),
        in_specs=[
            pl.BlockSpec((BB, DIM, PIX), lambda b: (b, 0, 0)),
            pl.BlockSpec((NUM_CODES, DIM), lambda b: (0, 0)),
        ],
        out_specs=[
            pl.BlockSpec((BB, 32, 32), lambda b: (b, 0, 0)),
            pl.BlockSpec((BB, DIM, PIX), lambda b: (b, 0, 0)),
        ],
        out_shape=[
            jax.ShapeDtypeStruct((B, 32, 32), jnp.int32),
            jax.ShapeDtypeStruct((B, DIM, PIX), jnp.float32),
        ],
        scratch_shapes=[
            pltpu.VMEM((NUM_CODES, DIM), jnp.float32),
            pltpu.VMEM((NUM_CODES, 1), jnp.float32),
            pltpu.VMEM((DIM, NUM_CODES), jnp.float32),
        ],
    )(x, codebook)
    return (zis, zqs.reshape(B, DIM, 32, 32))


# speedup vs baseline: 1.0054x; 1.0007x over previous
import jax
import jax.numpy as jnp
from jax import lax
from jax.experimental import pallas as pl
from jax.experimental.pallas import tpu as pltpu

NUM_CODES = 1024
DIM = 64
PIX = 1024


def _vq_body(x_ref, cb_ref, zis_ref, zqs_ref, cbn2_ref, c2_ref, cbt_ref):
    @pl.when(pl.program_id(0) == 0)
    def _init():
        cb0 = cb_ref[...]
        cbn2_ref[...] = cb0 * -2.0
        c2_ref[...] = jnp.sum(cb0 * cb0, axis=1, keepdims=True)
        cbt_ref[...] = cb0.T

    for s in range(x_ref.shape[0]):
        x = x_ref[s]

        mmn2 = lax.dot_general(cbn2_ref[...], x, (((1,), (0,)), ((), ())),
                               precision=lax.Precision.DEFAULT)
        z2 = jnp.sum(x * x, axis=0)
        dist = (z2[None, :] + c2_ref[...]) + mmn2

        idx = jnp.argmin(dist, axis=0)
        ii = lax.broadcasted_iota(jnp.int32, (NUM_CODES, PIX), 0)
        zis_ref[s] = idx.reshape(8, 128)

        onehot = (ii == idx[None, :]).astype(jnp.float32)
        zq = lax.dot_general(cbt_ref[...], onehot, (((1,), (0,)), ((), ())),
                             precision=lax.Precision.DEFAULT)
        zqs_ref[s] = zq


BB = 4


def kernel(inputs, codebook):
    B = inputs.shape[0]
    x = inputs.reshape(B, DIM, PIX)
    zis, zqs = pl.pallas_call(
        _vq_body,
        grid=(B // BB,),
        in_specs=[
            pl.BlockSpec((BB, DIM, PIX), lambda b: (b, 0, 0)),
            pl.BlockSpec((NUM_CODES, DIM), lambda b: (0, 0)),
        ],
        out_specs=[
            pl.BlockSpec((BB, 8, 128), lambda b: (b, 0, 0)),
            pl.BlockSpec((BB, DIM, PIX), lambda b: (b, 0, 0)),
        ],
        out_shape=[
            jax.ShapeDtypeStruct((B, 8, 128), jnp.int32),
            jax.ShapeDtypeStruct((B, DIM, PIX), jnp.float32),
        ],
        scratch_shapes=[
            pltpu.VMEM((NUM_CODES, DIM), jnp.float32),
            pltpu.VMEM((NUM_CODES, 1), jnp.float32),
            pltpu.VMEM((DIM, NUM_CODES), jnp.float32),
        ],
    )(x, codebook)
    return (zis.reshape(B, 32, 32), zqs.reshape(B, DIM, 32, 32))
